# baseline (speedup 1.0000x reference)
import functools

import jax
import jax.numpy as jnp
from jax import lax
from jax.experimental import pallas as pl
from jax.experimental.pallas import tpu as pltpu
from jax.experimental.pallas import tpu_sc as plsc

B, L, D = 16, 4096, 1024

_info = plsc.get_sparse_core_info()
_NC, _NS = _info.num_cores, _info.num_subcores


def _take_last_body(x_hbm, slen_hbm, out_hbm, slen_v, idx_v, rows_v, sem):
    c = lax.axis_index("c")
    s = lax.axis_index("s")
    j = s * _NC + c

    @pl.when(j == 0)
    def _():
        pltpu.sync_copy(slen_hbm, slen_v)
        sl = slen_v[...]
        iota = lax.iota(jnp.int32, B)
        idx_v[...] = iota * L + sl - 1
        pltpu.async_copy(x_hbm.at[idx_v], rows_v, sem).wait()
        pltpu.sync_copy(rows_v, out_hbm)


_take_last = functools.partial(
    pl.kernel,
    mesh=plsc.VectorSubcoreMesh(core_axis_name="c", subcore_axis_name="s"),
    out_type=jax.ShapeDtypeStruct((B, D), jnp.float32),
    scratch_types=[
        pltpu.VMEM((B,), jnp.int32),
        pltpu.VMEM((B,), jnp.int32),
        pltpu.VMEM((B, D), jnp.float32),
        pltpu.SemaphoreType.DMA,
    ],
)(_take_last_body)


@jax.jit
def kernel(x, seq_len):
    table = x.reshape(B * L, D)
    return _take_last(table, seq_len)


# speedup vs baseline: 1.0632x; 1.0632x over previous
import functools

import jax
import jax.numpy as jnp
from jax import lax
from jax.experimental import pallas as pl
from jax.experimental.pallas import tpu as pltpu
from jax.experimental.pallas import tpu_sc as plsc

B, L, D = 16, 4096, 1024

_info = plsc.get_sparse_core_info()
_NC, _NS = _info.num_cores, _info.num_subcores


def _take_last_body(x_hbm, slen_hbm, out_hbm, slen_v, idx_v, rows_v, sem):
    pltpu.sync_copy(slen_hbm, slen_v)
    sl = slen_v[...]
    iota = lax.iota(jnp.int32, B)
    idx_v[...] = iota * L + sl - 1
    pltpu.async_copy(x_hbm.at[idx_v], rows_v, sem).wait()
    pltpu.sync_copy(rows_v, out_hbm)


_take_last = functools.partial(
    pl.kernel,
    mesh=plsc.VectorSubcoreMesh(
        core_axis_name="c", subcore_axis_name="s", num_cores=1, num_subcores=1
    ),
    out_type=jax.ShapeDtypeStruct((B, D), jnp.float32),
    scratch_types=[
        pltpu.VMEM((B,), jnp.int32),
        pltpu.VMEM((B,), jnp.int32),
        pltpu.VMEM((B, D), jnp.float32),
        pltpu.SemaphoreType.DMA,
    ],
)(_take_last_body)


@jax.jit
def kernel(x, seq_len):
    table = x.reshape(B * L, D)
    return _take_last(table, seq_len)


# speedup vs baseline: 1.1113x; 1.0453x over previous
import functools

import jax
import jax.numpy as jnp
from jax import lax
from jax.experimental import pallas as pl
from jax.experimental.pallas import tpu as pltpu
from jax.experimental.pallas import tpu_sc as plsc

B, L, D = 16, 4096, 1024


def _take_last_body(x_hbm, slen_hbm, out_hbm, slen_s, sems):
    pltpu.sync_copy(slen_hbm, slen_s)
    for b in range(B):
        row = b * L + slen_s[b] - 1
        pltpu.async_copy(
            x_hbm.at[pl.ds(row, 1)], out_hbm.at[pl.ds(b, 1)], sems.at[b]
        )
    for b in range(B):
        pltpu.make_async_copy(
            x_hbm.at[pl.ds(0, 1)], out_hbm.at[pl.ds(b, 1)], sems.at[b]
        ).wait()


_take_last = functools.partial(
    pl.kernel,
    mesh=plsc.ScalarSubcoreMesh(axis_name="c", num_cores=1),
    out_type=jax.ShapeDtypeStruct((B, D), jnp.float32),
    scratch_types=[
        pltpu.SMEM((B,), jnp.int32),
        pltpu.SemaphoreType.DMA((B,)),
    ],
)(_take_last_body)


@jax.jit
def kernel(x, seq_len):
    table = x.reshape(B * L, D)
    return _take_last(table, seq_len)


# speedup vs baseline: 1.1155x; 1.0037x over previous
import functools

import jax
import jax.numpy as jnp
from jax import lax
from jax.experimental import pallas as pl
from jax.experimental.pallas import tpu as pltpu
from jax.experimental.pallas import tpu_sc as plsc

B, L, D = 16, 4096, 1024


def _take_last_body(x_hbm, slen_hbm, out_hbm, slen_s, sem):
    pltpu.sync_copy(slen_hbm, slen_s)

    def issue(b, carry):
        row = b * L + slen_s[b] - 1
        pltpu.async_copy(x_hbm.at[pl.ds(row, 1)], out_hbm.at[pl.ds(b, 1)], sem)
        return carry

    lax.fori_loop(0, B, issue, 0)

    def drain(b, carry):
        pltpu.make_async_copy(
            x_hbm.at[pl.ds(0, 1)], out_hbm.at[pl.ds(b, 1)], sem
        ).wait()
        return carry

    lax.fori_loop(0, B, drain, 0)


_take_last = functools.partial(
    pl.kernel,
    mesh=plsc.ScalarSubcoreMesh(axis_name="c", num_cores=1),
    out_type=jax.ShapeDtypeStruct((B, D), jnp.float32),
    scratch_types=[
        pltpu.SMEM((B,), jnp.int32),
        pltpu.SemaphoreType.DMA,
    ],
)(_take_last_body)


@jax.jit
def kernel(x, seq_len):
    table = x.reshape(B * L, D)
    return _take_last(table, seq_len)


# speedup vs baseline: 2.6011x; 2.3318x over previous
import jax
import jax.numpy as jnp
from jax.experimental import pallas as pl
from jax.experimental.pallas import tpu as pltpu

B, L, D = 16, 4096, 1024


def _take_last_body(slen_ref, x_ref, out_ref):
    b = pl.program_id(0)
    r = (slen_ref[b] - 1) % 8
    out_ref[pl.ds(b % 8, 1), :] = x_ref[0, pl.ds(r, 1), :]


_take_last = pl.pallas_call(
    _take_last_body,
    grid_spec=pltpu.PrefetchScalarGridSpec(
        num_scalar_prefetch=1,
        grid=(B,),
        in_specs=[
            pl.BlockSpec((1, 8, D), lambda b, slen: (b, (slen[b] - 1) // 8, 0)),
        ],
        out_specs=pl.BlockSpec((8, D), lambda b, slen: (b // 8, 0)),
    ),
    out_shape=jax.ShapeDtypeStruct((B, D), jnp.float32),
)


@jax.jit
def kernel(x, seq_len):
    return _take_last(seq_len, x)


# speedup vs baseline: 9.3733x; 3.6036x over previous
import jax
import jax.numpy as jnp
from jax.experimental import pallas as pl
from jax.experimental.pallas import tpu as pltpu

B, L, D = 16, 4096, 1024


def _take_last_body(slen_ref, x_ref, out_ref, rows, sem, osem):
    for b in range(B):
        row = slen_ref[b] - 1
        pltpu.make_async_copy(
            x_ref.at[b, pl.ds(row, 1)], rows.at[pl.ds(b, 1)], sem
        ).start()
    for b in range(B):
        pltpu.make_async_copy(
            x_ref.at[b, pl.ds(0, 1)], rows.at[pl.ds(b, 1)], sem
        ).wait()
    pltpu.make_async_copy(rows, out_ref, osem).start()
    pltpu.make_async_copy(rows, out_ref, osem).wait()


_take_last = pl.pallas_call(
    _take_last_body,
    out_shape=jax.ShapeDtypeStruct((B, D), jnp.float32),
    in_specs=[
        pl.BlockSpec(memory_space=pltpu.SMEM),
        pl.BlockSpec(memory_space=pl.ANY),
    ],
    out_specs=pl.BlockSpec(memory_space=pl.ANY),
    scratch_shapes=[
        pltpu.VMEM((B, D), jnp.float32),
        pltpu.SemaphoreType.DMA,
        pltpu.SemaphoreType.DMA,
    ],
)


@jax.jit
def kernel(x, seq_len):
    return _take_last(seq_len, x)


# speedup vs baseline: 9.5481x; 1.0186x over previous
import jax
import jax.numpy as jnp
from jax.experimental import pallas as pl
from jax.experimental.pallas import tpu as pltpu

B, L, D = 16, 4096, 1024


def _take_last_body(slen_ref, x_ref, out_ref, rows, sem, osem):
    for b in range(B):
        row = slen_ref[b] - 1
        pltpu.make_async_copy(
            x_ref.at[b, pl.ds(row, 1)], rows.at[pl.ds(b, 1)], sem
        ).start()
    for b in range(B):
        pltpu.make_async_copy(
            x_ref.at[b, pl.ds(0, 1)], rows.at[pl.ds(b, 1)], sem
        ).wait()
        pltpu.make_async_copy(
            rows.at[pl.ds(b, 1)], out_ref.at[pl.ds(b, 1)], osem
        ).start()
    pltpu.make_async_copy(rows, out_ref, osem).wait()


_take_last = pl.pallas_call(
    _take_last_body,
    out_shape=jax.ShapeDtypeStruct((B, D), jnp.float32),
    in_specs=[
        pl.BlockSpec(memory_space=pltpu.SMEM),
        pl.BlockSpec(memory_space=pl.ANY),
    ],
    out_specs=pl.BlockSpec(memory_space=pl.ANY),
    scratch_shapes=[
        pltpu.VMEM((B, D), jnp.float32),
        pltpu.SemaphoreType.DMA,
        pltpu.SemaphoreType.DMA,
    ],
)


@jax.jit
def kernel(x, seq_len):
    return _take_last(seq_len, x)
